# baseline (speedup 1.0000x reference)
import functools

import jax
import jax.numpy as jnp
from jax import lax
from jax.experimental import pallas as pl
from jax.experimental.pallas import tpu as pltpu
from jax.experimental.pallas import tpu_sc as plsc

_NC = 2
_NS = 16
_NW = _NC * _NS
_L = 16
_CH = 4096
_CHV = _CH // _L
_D = 10


def _sc_body(x_hbm, bins_hbm, out_hbm,
             xb0, xb1, pb0, pb1, ib0, ib1, onesb, binsb, zb,
             si0, si1, so0, so1):
    M = x_hbm.shape[0]
    rows_per_w = M // _NW
    wid = lax.axis_index("s") * _NC + lax.axis_index("c")
    rbase = wid * rows_per_w

    pltpu.sync_copy(bins_hbm, binsb.at[pl.ds(0, 10)])
    vb = binsb[pl.ds(0, _L)]
    thr = [jnp.broadcast_to(vb[k], (_L,)) for k in range(1, 9)]
    vone = jnp.full((_L,), 1.0, jnp.float32)
    vzero = jnp.zeros((_L,), jnp.float32)
    viota = lax.iota(jnp.int32, _L)

    def _init(v, _):
        onesb[pl.ds(v * _L, _L)] = vone
        return 0
    lax.fori_loop(0, _CHV, _init, 0)
    vo = onesb[pl.ds(0, _L)]
    zb[pl.ds(0, _L)] = vzero
    vz = zb[pl.ds(0, _L)]

    def _zero(v, _):
        pb0[pl.ds(v * _L, _L)] = vzero
        pb1[pl.ds(v * _L, _L)] = vzero
        return 0
    lax.fori_loop(0, 9 * _CHV, _zero, 0)

    def _in_copy(xb, si, r, off):
        return pltpu.make_async_copy(x_hbm.at[r, pl.ds(off, _CH)], xb, si)

    def _out_copies(pbk, so, r, off):
        cps = [pltpu.make_async_copy(pbk.at[pl.ds(c * _CH, _CH)],
                                     out_hbm.at[c + 1, r, pl.ds(off, _CH)],
                                     so)
               for c in range(9)]
        cps.append(pltpu.make_async_copy(onesb, out_hbm.at[0, r, pl.ds(off, _CH)], so))
        return cps

    _in_copy(xb0, si0, rbase, 0).start()

    def _chunk(r, k, xb, pbk, ibk, si, so):
        off = k * _CH

        if k == 0:
            _in_copy(xb1, si1, r, _CH).start()
        else:
            @pl.when(r + 1 < rbase + rows_per_w)
            def _():
                _in_copy(xb0, si0, r + 1, 0).start()

        @pl.when(r > rbase)
        def _():
            for cp in _out_copies(pbk, so, r, off):
                cp.wait()

            def _clear(v, _):
                sidx = ibk[pl.ds(v * _L, _L)]
                plsc.store_scatter(pbk, [sidx], vz)
                return 0
            lax.fori_loop(0, _CHV, _clear, 0)

        _in_copy(xb, si, r, off).wait()

        def _compute(v, _):
            xv = xb[pl.ds(v * _L, _L)]
            acc = jnp.where(xv > thr[0], 1, 0)
            for t in thr[1:]:
                acc = acc + jnp.where(xv > t, 1, 0)
            sidx = acc * _CH + (v * _L + viota)
            ibk[pl.ds(v * _L, _L)] = sidx
            plsc.store_scatter(pbk, [sidx], vo)
            return 0
        lax.fori_loop(0, _CHV, _compute, 0)

        for cp in _out_copies(pbk, so, r, off):
            cp.start()

    def _row(r, _):
        _chunk(r, 0, xb0, pb0, ib0, si0, so0)
        _chunk(r, 1, xb1, pb1, ib1, si1, so1)
        return 0
    lax.fori_loop(rbase, rbase + rows_per_w, _row, 0)

    last = rbase + rows_per_w - 1
    for cp in _out_copies(pb0, so0, last, 0):
        cp.wait()
    for cp in _out_copies(pb1, so1, last, _CH):
        cp.wait()


def kernel(x, bins):
    M, N = x.shape
    mesh = plsc.VectorSubcoreMesh(core_axis_name="c", subcore_axis_name="s")
    run = pl.kernel(
        _sc_body,
        out_type=jax.ShapeDtypeStruct((_D, M, N), jnp.float32),
        mesh=mesh,
        scratch_types=[
            pltpu.VMEM((_CH,), jnp.float32),
            pltpu.VMEM((_CH,), jnp.float32),
            pltpu.VMEM((9 * _CH,), jnp.float32),
            pltpu.VMEM((9 * _CH,), jnp.float32),
            pltpu.VMEM((_CH,), jnp.int32),
            pltpu.VMEM((_CH,), jnp.int32),
            pltpu.VMEM((_CH,), jnp.float32),
            pltpu.VMEM((16,), jnp.float32),
            pltpu.VMEM((16,), jnp.float32),
            pltpu.SemaphoreType.DMA,
            pltpu.SemaphoreType.DMA,
            pltpu.SemaphoreType.DMA,
            pltpu.SemaphoreType.DMA,
        ],
        compiler_params=pltpu.CompilerParams(needs_layout_passes=False),
    )
    out = run(x, bins)
    return jnp.transpose(out, (1, 2, 0))


# speedup vs baseline: 2.7996x; 2.7996x over previous
import jax
import jax.numpy as jnp
from jax.experimental import pallas as pl
from jax.experimental.pallas import tpu as pltpu

_R = 64
_D = 10


def _body(bins_ref, x_ref, o_ref):
    x = x_ref[...]
    one = jnp.ones(x.shape, jnp.float32)
    zero = jnp.zeros(x.shape, jnp.float32)
    o_ref[0, :, :] = one
    above = [x > bins_ref[k] for k in range(1, 9)]
    o_ref[1, :, :] = jnp.where(above[0], zero, one)
    for c in range(2, 9):
        o_ref[c, :, :] = jnp.where(above[c - 2] & (~above[c - 1]), one, zero)
    o_ref[9, :, :] = jnp.where(above[7], one, zero)


def kernel(x, bins):
    M, N = x.shape
    grid = (M // _R,)
    out = pl.pallas_call(
        _body,
        grid=grid,
        in_specs=[
            pl.BlockSpec(memory_space=pltpu.SMEM),
            pl.BlockSpec((_R, N), lambda i: (i, 0)),
        ],
        out_specs=pl.BlockSpec((_D, _R, N), lambda i: (0, i, 0)),
        out_shape=jax.ShapeDtypeStruct((_D, M, N), jnp.float32),
        compiler_params=pltpu.CompilerParams(
            dimension_semantics=("parallel",),
        ),
    )(bins, x)
    return jnp.transpose(out, (1, 2, 0))
